# Initial kernel scaffold; baseline (speedup 1.0000x reference)
import jax
import jax.numpy as jnp
from jax.experimental import pallas as pl


def kernel(x, M, W_q, W_e, b_e, W_a, b_a, W_o, r_gate, ln_w, ln_b):
    raise NotImplementedError("write your pallas kernel here")


# speedup vs baseline: 7.3779x; 7.3779x over previous
import jax
import jax.numpy as jnp
from jax.experimental import pallas as pl

B, L, D = 4, 2048, 1024
SLOTS = 1024
TOP_K = 8
SCALE = D ** (-0.5)
EPS = 1e-5

TL = 256


def _topk_threshold(s, axis):
    cur = s
    cum = jnp.zeros_like(jnp.max(s, axis=axis, keepdims=True))
    kth = jnp.full_like(cum, -jnp.inf)
    for _ in range(TOP_K):
        m = jnp.max(cur, axis=axis, keepdims=True)
        c = jnp.sum((s == m).astype(s.dtype), axis=axis, keepdims=True)
        take = jnp.logical_and(cum < TOP_K, cum + c >= TOP_K)
        kth = jnp.where(take, m, kth)
        cum = cum + c
        cur = jnp.where(cur == m, -jnp.inf, cur)
    return kth


def _sparse_softmax(s, axis):
    kth = _topk_threshold(s, axis)
    m1 = jnp.max(s, axis=axis, keepdims=True)
    e = jnp.where(s >= kth, jnp.exp(s - m1), 0.0)
    return e / jnp.sum(e, axis=axis, keepdims=True)


def _read_kernel(x_ref, M_ref, Wq_ref, Wo_ref, gate_ref, lnw_ref, lnb_ref,
                 out_ref, rep_ref):
    x = x_ref[0]
    Mb = M_ref[0]
    cdims = (((1,), (1,)), ((), ()))
    q = jax.lax.dot_general(x, Wq_ref[...], cdims,
                            preferred_element_type=jnp.float32)
    s = jax.lax.dot_general(q, Mb, cdims,
                            preferred_element_type=jnp.float32) * SCALE
    addr = _sparse_softmax(s, axis=1)
    r = jnp.dot(addr, Mb, preferred_element_type=jnp.float32)
    mu = jnp.mean(r, axis=1, keepdims=True)
    var = jnp.mean((r - mu) ** 2, axis=1, keepdims=True)
    ln = (r - mu) * jax.lax.rsqrt(var + EPS) * lnw_ref[...] + lnb_ref[...]
    out_ref[0] = jax.lax.dot_general(ln, Wo_ref[...], cdims,
                                     preferred_element_type=jnp.float32)
    rep_ref[0] = jax.nn.sigmoid(gate_ref[...]) * r


def _write_kernel(xl_ref, M_ref, Wq_ref, We_ref, be_ref, Wa_ref, ba_ref,
                  Mout_ref):
    xl = xl_ref[0]
    Mb = M_ref[0]
    cdims = (((1,), (1,)), ((), ()))
    q = jax.lax.dot_general(xl, Wq_ref[...], cdims,
                            preferred_element_type=jnp.float32)
    s = jax.lax.dot_general(Mb, q, cdims,
                            preferred_element_type=jnp.float32) * SCALE
    addr = _sparse_softmax(s, axis=0)
    erase = jax.nn.sigmoid(
        jax.lax.dot_general(xl, We_ref[...], cdims,
                            preferred_element_type=jnp.float32) + be_ref[...])
    add = jnp.tanh(
        jax.lax.dot_general(xl, Wa_ref[...], cdims,
                            preferred_element_type=jnp.float32) + ba_ref[...])
    Mout_ref[0] = Mb * (1.0 - addr * erase) + addr * add


def kernel(x, M, W_q, W_e, b_e, W_a, b_a, W_o, r_gate, ln_w, ln_b):
    gate2 = r_gate.reshape(1, D)
    lnw2 = ln_w.reshape(1, D)
    lnb2 = ln_b.reshape(1, D)
    be2 = b_e.reshape(1, D)
    ba2 = b_a.reshape(1, D)
    x_last = x[:, -1].reshape(B, 1, D)

    out, rep = pl.pallas_call(
        _read_kernel,
        grid=(B, L // TL),
        in_specs=[
            pl.BlockSpec((1, TL, D), lambda b, l: (b, l, 0)),
            pl.BlockSpec((1, SLOTS, D), lambda b, l: (b, 0, 0)),
            pl.BlockSpec((D, D), lambda b, l: (0, 0)),
            pl.BlockSpec((D, D), lambda b, l: (0, 0)),
            pl.BlockSpec((1, D), lambda b, l: (0, 0)),
            pl.BlockSpec((1, D), lambda b, l: (0, 0)),
            pl.BlockSpec((1, D), lambda b, l: (0, 0)),
        ],
        out_specs=[
            pl.BlockSpec((1, TL, D), lambda b, l: (b, l, 0)),
            pl.BlockSpec((1, TL, D), lambda b, l: (b, l, 0)),
        ],
        out_shape=[
            jax.ShapeDtypeStruct((B, L, D), jnp.float32),
            jax.ShapeDtypeStruct((B, L, D), jnp.float32),
        ],
    )(x, M, W_q, W_o, gate2, lnw2, lnb2)

    M_new = pl.pallas_call(
        _write_kernel,
        grid=(B,),
        in_specs=[
            pl.BlockSpec((1, 1, D), lambda b: (b, 0, 0)),
            pl.BlockSpec((1, SLOTS, D), lambda b: (b, 0, 0)),
            pl.BlockSpec((D, D), lambda b: (0, 0)),
            pl.BlockSpec((D, D), lambda b: (0, 0)),
            pl.BlockSpec((1, D), lambda b: (0, 0)),
            pl.BlockSpec((D, D), lambda b: (0, 0)),
            pl.BlockSpec((1, D), lambda b: (0, 0)),
        ],
        out_specs=pl.BlockSpec((1, SLOTS, D), lambda b: (b, 0, 0)),
        out_shape=jax.ShapeDtypeStruct((B, SLOTS, D), jnp.float32),
    )(x_last, M, W_q, W_e, be2, W_a, ba2)

    return out, rep, M_new


# speedup vs baseline: 7.9235x; 1.0739x over previous
import jax
import jax.numpy as jnp
from jax.experimental import pallas as pl

B, L, D = 4, 2048, 1024
SLOTS = 1024
TOP_K = 8
SCALE = D ** (-0.5)
EPS = 1e-5

TL = 256
NL = L // TL
CDIMS = (((1,), (1,)), ((), ()))


def _sparse_softmax(s, axis):
    cur = s
    m1 = None
    cum = None
    kth = None
    for i in range(TOP_K):
        m = jnp.max(cur, axis=axis, keepdims=True)
        eq = cur == m
        c = jnp.sum(eq.astype(s.dtype), axis=axis, keepdims=True)
        if i == 0:
            m1 = m
            kth = m
            cum = c
        else:
            take = jnp.logical_and(cum < TOP_K, cum + c >= TOP_K)
            kth = jnp.where(take, m, kth)
            cum = cum + c
        if i < TOP_K - 1:
            cur = jnp.where(eq, -jnp.inf, cur)
    e = jnp.where(s >= kth, jnp.exp(s - m1), 0.0)
    return e / jnp.sum(e, axis=axis, keepdims=True)


def _fused_kernel(x_ref, M_ref, Wq_ref, Wo_ref, We_ref, Wa_ref, gate_ref,
                  lnw_ref, lnb_ref, be_ref, ba_ref,
                  out_ref, rep_ref, Mnew_ref):
    x = x_ref[0]
    Mb = M_ref[0]
    q = jax.lax.dot_general(x, Wq_ref[...], CDIMS,
                            preferred_element_type=jnp.float32)
    s = jax.lax.dot_general(q, Mb, CDIMS,
                            preferred_element_type=jnp.float32) * SCALE
    addr = _sparse_softmax(s, axis=1)
    r = jnp.dot(addr, Mb, preferred_element_type=jnp.float32)
    mu = jnp.mean(r, axis=1, keepdims=True)
    var = jnp.mean((r - mu) ** 2, axis=1, keepdims=True)
    ln = (r - mu) * jax.lax.rsqrt(var + EPS) * lnw_ref[...] + lnb_ref[...]
    out_ref[0] = jax.lax.dot_general(ln, Wo_ref[...], CDIMS,
                                     preferred_element_type=jnp.float32)
    rep_ref[0] = jax.nn.sigmoid(gate_ref[...]) * r

    @pl.when(pl.program_id(1) == NL - 1)
    def _write_path():
        q_last = q[TL - 1:TL, :]
        s_col = jax.lax.dot_general(Mb, q_last, CDIMS,
                                    preferred_element_type=jnp.float32) * SCALE
        addr_w = _sparse_softmax(s_col, axis=0)
        xl = x[TL - 1:TL, :]
        erase = jax.nn.sigmoid(
            jax.lax.dot_general(xl, We_ref[...], CDIMS,
                                preferred_element_type=jnp.float32)
            + be_ref[...])
        add = jnp.tanh(
            jax.lax.dot_general(xl, Wa_ref[...], CDIMS,
                                preferred_element_type=jnp.float32)
            + ba_ref[...])
        Mnew_ref[0] = Mb * (1.0 - addr_w * erase) + addr_w * add


def kernel(x, M, W_q, W_e, b_e, W_a, b_a, W_o, r_gate, ln_w, ln_b):
    gate2 = r_gate.reshape(1, D)
    lnw2 = ln_w.reshape(1, D)
    lnb2 = ln_b.reshape(1, D)
    be2 = b_e.reshape(1, D)
    ba2 = b_a.reshape(1, D)

    wcell = lambda b, l: (0, 0)
    out, rep, M_new = pl.pallas_call(
        _fused_kernel,
        grid=(B, NL),
        in_specs=[
            pl.BlockSpec((1, TL, D), lambda b, l: (b, l, 0)),
            pl.BlockSpec((1, SLOTS, D), lambda b, l: (b, 0, 0)),
            pl.BlockSpec((D, D), wcell),
            pl.BlockSpec((D, D), wcell),
            pl.BlockSpec((D, D), wcell),
            pl.BlockSpec((D, D), wcell),
            pl.BlockSpec((1, D), wcell),
            pl.BlockSpec((1, D), wcell),
            pl.BlockSpec((1, D), wcell),
            pl.BlockSpec((1, D), wcell),
            pl.BlockSpec((1, D), wcell),
        ],
        out_specs=[
            pl.BlockSpec((1, TL, D), lambda b, l: (b, l, 0)),
            pl.BlockSpec((1, TL, D), lambda b, l: (b, l, 0)),
            pl.BlockSpec((1, SLOTS, D), lambda b, l: (b, 0, 0)),
        ],
        out_shape=[
            jax.ShapeDtypeStruct((B, L, D), jnp.float32),
            jax.ShapeDtypeStruct((B, L, D), jnp.float32),
            jax.ShapeDtypeStruct((B, SLOTS, D), jnp.float32),
        ],
    )(x, M, W_q, W_o, W_e, W_a, gate2, lnw2, lnb2, be2, ba2)

    return out, rep, M_new


# speedup vs baseline: 8.9026x; 1.1236x over previous
import jax
import jax.numpy as jnp
from jax.experimental import pallas as pl

B, L, D = 4, 2048, 1024
SLOTS = 1024
TOP_K = 8
SCALE = D ** (-0.5)
EPS = 1e-5

TL = 256
NL = L // TL
CDIMS = (((1,), (1,)), ((), ()))


def _sparse_softmax(s, axis):
    cur = s
    m1 = None
    cum = None
    kth = None
    for i in range(TOP_K):
        m = jnp.max(cur, axis=axis, keepdims=True)
        eq = cur == m
        c = jnp.sum(eq.astype(s.dtype), axis=axis, keepdims=True)
        if i == 0:
            m1 = m
            kth = m
            cum = c
        else:
            take = jnp.logical_and(cum < TOP_K, cum + c >= TOP_K)
            kth = jnp.where(take, m, kth)
            cum = cum + c
        if i < TOP_K - 1:
            cur = jnp.where(eq, -jnp.inf, cur)
    e = jnp.where(s >= kth, jnp.exp(s - m1), 0.0)
    return e / jnp.sum(e, axis=axis, keepdims=True)


_CES = ((0, 1), (2, 3), (4, 5), (6, 7),
        (0, 2), (1, 3), (4, 6), (5, 7),
        (1, 2), (5, 6),
        (0, 4), (1, 5), (2, 6), (3, 7),
        (1, 4), (3, 6),
        (2, 4), (3, 5),
        (3, 4))
_NPARTS = SLOTS // 128


def _sparse_softmax_rows(s):
    parts = [s[:, j * 128:(j + 1) * 128] for j in range(_NPARTS)]
    for i, j in _CES:
        a, b = parts[i], parts[j]
        parts[i] = jnp.maximum(a, b)
        parts[j] = jnp.minimum(a, b)
    m1 = None
    cum = None
    kth = None
    for i in range(TOP_K):
        front = parts[0]
        m = jnp.max(front, axis=1, keepdims=True)
        eq = front == m
        c = jnp.sum(eq.astype(s.dtype), axis=1, keepdims=True)
        if i == 0:
            m1 = m
            kth = m
            cum = c
        else:
            take = jnp.logical_and(cum < TOP_K, cum + c >= TOP_K)
            kth = jnp.where(take, m, kth)
            cum = cum + c
        for j in range(_NPARTS - 1 - i):
            parts[j] = jnp.where(eq, parts[j + 1], parts[j])
        if i < TOP_K - 1:
            parts[_NPARTS - 1 - i] = jnp.where(eq, -jnp.inf,
                                               parts[_NPARTS - 1 - i])
    e = jnp.where(s >= kth, jnp.exp(s - m1), 0.0)
    return e / jnp.sum(e, axis=1, keepdims=True)


def _fused_kernel(x_ref, M_ref, Wq_ref, Wo_ref, We_ref, Wa_ref, gate_ref,
                  lnw_ref, lnb_ref, be_ref, ba_ref,
                  out_ref, rep_ref, Mnew_ref):
    x = x_ref[0]
    Mb = M_ref[0]
    q = jax.lax.dot_general(x, Wq_ref[...], CDIMS,
                            preferred_element_type=jnp.float32)
    s = jax.lax.dot_general(q, Mb, CDIMS,
                            preferred_element_type=jnp.float32) * SCALE
    addr = _sparse_softmax_rows(s)
    r = jnp.dot(addr, Mb, preferred_element_type=jnp.float32)
    mu = jnp.mean(r, axis=1, keepdims=True)
    var = jnp.mean((r - mu) ** 2, axis=1, keepdims=True)
    ln = (r - mu) * jax.lax.rsqrt(var + EPS) * lnw_ref[...] + lnb_ref[...]
    out_ref[0] = jax.lax.dot_general(ln, Wo_ref[...], CDIMS,
                                     preferred_element_type=jnp.float32)
    rep_ref[0] = jax.nn.sigmoid(gate_ref[...]) * r

    @pl.when(pl.program_id(1) == NL - 1)
    def _write_path():
        q_last = q[TL - 1:TL, :]
        s_col = jax.lax.dot_general(Mb, q_last, CDIMS,
                                    preferred_element_type=jnp.float32) * SCALE
        addr_w = _sparse_softmax(s_col, axis=0)
        xl = x[TL - 1:TL, :]
        erase = jax.nn.sigmoid(
            jax.lax.dot_general(xl, We_ref[...], CDIMS,
                                preferred_element_type=jnp.float32)
            + be_ref[...])
        add = jnp.tanh(
            jax.lax.dot_general(xl, Wa_ref[...], CDIMS,
                                preferred_element_type=jnp.float32)
            + ba_ref[...])
        Mnew_ref[0] = Mb * (1.0 - addr_w * erase) + addr_w * add


def kernel(x, M, W_q, W_e, b_e, W_a, b_a, W_o, r_gate, ln_w, ln_b):
    gate2 = r_gate.reshape(1, D)
    lnw2 = ln_w.reshape(1, D)
    lnb2 = ln_b.reshape(1, D)
    be2 = b_e.reshape(1, D)
    ba2 = b_a.reshape(1, D)

    wcell = lambda b, l: (0, 0)
    out, rep, M_new = pl.pallas_call(
        _fused_kernel,
        grid=(B, NL),
        in_specs=[
            pl.BlockSpec((1, TL, D), lambda b, l: (b, l, 0)),
            pl.BlockSpec((1, SLOTS, D), lambda b, l: (b, 0, 0)),
            pl.BlockSpec((D, D), wcell),
            pl.BlockSpec((D, D), wcell),
            pl.BlockSpec((D, D), wcell),
            pl.BlockSpec((D, D), wcell),
            pl.BlockSpec((1, D), wcell),
            pl.BlockSpec((1, D), wcell),
            pl.BlockSpec((1, D), wcell),
            pl.BlockSpec((1, D), wcell),
            pl.BlockSpec((1, D), wcell),
        ],
        out_specs=[
            pl.BlockSpec((1, TL, D), lambda b, l: (b, l, 0)),
            pl.BlockSpec((1, TL, D), lambda b, l: (b, l, 0)),
            pl.BlockSpec((1, SLOTS, D), lambda b, l: (b, 0, 0)),
        ],
        out_shape=[
            jax.ShapeDtypeStruct((B, L, D), jnp.float32),
            jax.ShapeDtypeStruct((B, L, D), jnp.float32),
            jax.ShapeDtypeStruct((B, SLOTS, D), jnp.float32),
        ],
    )(x, M, W_q, W_o, W_e, W_a, gate2, lnw2, lnb2, be2, ba2)

    return out, rep, M_new


# speedup vs baseline: 9.0167x; 1.0128x over previous
import jax
import jax.numpy as jnp
from jax.experimental import pallas as pl

B, L, D = 4, 2048, 1024
SLOTS = 1024
TOP_K = 8
SCALE = D ** (-0.5)
EPS = 1e-5

TL = 256
NL = L // TL
CDIMS = (((1,), (1,)), ((), ()))

_CES = ((0, 1), (2, 3), (4, 5), (6, 7),
        (0, 2), (1, 3), (4, 6), (5, 7),
        (1, 2), (5, 6),
        (0, 4), (1, 5), (2, 6), (3, 7),
        (1, 4), (3, 6),
        (2, 4), (3, 5),
        (3, 4))
_NPARTS = SLOTS // 128


def _bf16(a):
    return a.astype(jnp.bfloat16)


def _topk_rows(s):
    parts = [s[:, j * 128:(j + 1) * 128] for j in range(_NPARTS)]
    for i, j in _CES:
        a, b = parts[i], parts[j]
        parts[i] = jnp.maximum(a, b)
        parts[j] = jnp.minimum(a, b)
    m1 = None
    cum = None
    kth = None
    for i in range(TOP_K):
        front = parts[0]
        m = jnp.max(front, axis=1, keepdims=True)
        eq = front == m
        c = jnp.sum(eq.astype(s.dtype), axis=1, keepdims=True)
        if i == 0:
            m1 = m
            kth = m
            cum = c
        else:
            take = jnp.logical_and(cum < TOP_K, cum + c >= TOP_K)
            kth = jnp.where(take, m, kth)
            cum = cum + c
        for j in range(_NPARTS - 1 - i):
            parts[j] = jnp.where(eq, parts[j + 1], parts[j])
        if i < TOP_K - 1:
            parts[_NPARTS - 1 - i] = jnp.where(eq, -jnp.inf,
                                               parts[_NPARTS - 1 - i])
    return kth, m1


def _sparse_softmax_rows(s):
    kth, m1 = _topk_rows(s)
    e = jnp.where(s >= kth, jnp.exp(s - m1), 0.0)
    return e / jnp.sum(e, axis=1, keepdims=True)


def _fused_kernel(x_ref, M_ref, Wq_ref, Wo_ref, We_ref, Wa_ref, gate_ref,
                  lnw_ref, lnb_ref, be_ref, ba_ref,
                  out_ref, rep_ref, Mnew_ref):
    x = x_ref[0]
    Mb = M_ref[0]
    Mb16 = _bf16(Mb)
    q = jax.lax.dot_general(x, Wq_ref[...], CDIMS,
                            preferred_element_type=jnp.float32)
    s = jax.lax.dot_general(q, Mb, CDIMS,
                            preferred_element_type=jnp.float32) * SCALE
    addr = _sparse_softmax_rows(s)
    r = jax.lax.dot_general(_bf16(addr), Mb16, (((1,), (0,)), ((), ())),
                            preferred_element_type=jnp.float32)
    mu = jnp.mean(r, axis=1, keepdims=True)
    var = jnp.mean((r - mu) ** 2, axis=1, keepdims=True)
    ln = (r - mu) * jax.lax.rsqrt(var + EPS) * lnw_ref[...] + lnb_ref[...]
    out_ref[0] = jax.lax.dot_general(_bf16(ln), _bf16(Wo_ref[...]), CDIMS,
                                     preferred_element_type=jnp.float32)
    rep_ref[0] = jax.nn.sigmoid(gate_ref[...]) * r

    @pl.when(pl.program_id(1) == NL - 1)
    def _write_path():
        q_last = q[TL - 1:TL, :]
        s_col = jax.lax.dot_general(Mb, q_last, CDIMS,
                                    preferred_element_type=jnp.float32) * SCALE
        kth, m1 = _topk_rows(s_col.T)
        e = jnp.where(s_col >= kth, jnp.exp(s_col - m1), 0.0)
        addr_w = e / jnp.sum(e, axis=0, keepdims=True)
        xl = x[TL - 1:TL, :]
        xl16 = _bf16(xl)
        erase = jax.nn.sigmoid(
            jax.lax.dot_general(xl16, _bf16(We_ref[...]), CDIMS,
                                preferred_element_type=jnp.float32)
            + be_ref[...])
        add = jnp.tanh(
            jax.lax.dot_general(xl16, _bf16(Wa_ref[...]), CDIMS,
                                preferred_element_type=jnp.float32)
            + ba_ref[...])
        Mnew_ref[0] = Mb * (1.0 - addr_w * erase) + addr_w * add


def kernel(x, M, W_q, W_e, b_e, W_a, b_a, W_o, r_gate, ln_w, ln_b):
    gate2 = r_gate.reshape(1, D)
    lnw2 = ln_w.reshape(1, D)
    lnb2 = ln_b.reshape(1, D)
    be2 = b_e.reshape(1, D)
    ba2 = b_a.reshape(1, D)

    wcell = lambda b, l: (0, 0)
    out, rep, M_new = pl.pallas_call(
        _fused_kernel,
        grid=(B, NL),
        in_specs=[
            pl.BlockSpec((1, TL, D), lambda b, l: (b, l, 0)),
            pl.BlockSpec((1, SLOTS, D), lambda b, l: (b, 0, 0)),
            pl.BlockSpec((D, D), wcell),
            pl.BlockSpec((D, D), wcell),
            pl.BlockSpec((D, D), wcell),
            pl.BlockSpec((D, D), wcell),
            pl.BlockSpec((1, D), wcell),
            pl.BlockSpec((1, D), wcell),
            pl.BlockSpec((1, D), wcell),
            pl.BlockSpec((1, D), wcell),
            pl.BlockSpec((1, D), wcell),
        ],
        out_specs=[
            pl.BlockSpec((1, TL, D), lambda b, l: (b, l, 0)),
            pl.BlockSpec((1, TL, D), lambda b, l: (b, l, 0)),
            pl.BlockSpec((1, SLOTS, D), lambda b, l: (b, 0, 0)),
        ],
        out_shape=[
            jax.ShapeDtypeStruct((B, L, D), jnp.float32),
            jax.ShapeDtypeStruct((B, L, D), jnp.float32),
            jax.ShapeDtypeStruct((B, SLOTS, D), jnp.float32),
        ],
    )(x, M, W_q, W_o, W_e, W_a, gate2, lnw2, lnb2, be2, ba2)

    return out, rep, M_new


# speedup vs baseline: 10.2407x; 1.1357x over previous
import jax
import jax.numpy as jnp
from jax.experimental import pallas as pl
from jax.experimental.pallas import tpu as pltpu

B, L, D = 4, 2048, 1024
SLOTS = 1024
TOP_K = 8
SCALE = D ** (-0.5)
EPS = 1e-5

TL = 512
NL = L // TL
MS = SLOTS // NL
CDIMS = (((1,), (1,)), ((), ()))

_CES = ((0, 1), (2, 3), (4, 5), (6, 7),
        (0, 2), (1, 3), (4, 6), (5, 7),
        (1, 2), (5, 6),
        (0, 4), (1, 5), (2, 6), (3, 7),
        (1, 4), (3, 6),
        (2, 4), (3, 5),
        (3, 4))
_NPARTS = SLOTS // 128


def _bf16(a):
    return a.astype(jnp.bfloat16)


def _topk_rows(s):
    parts = [s[:, j * 128:(j + 1) * 128] for j in range(_NPARTS)]
    for i, j in _CES:
        a, b = parts[i], parts[j]
        parts[i] = jnp.maximum(a, b)
        parts[j] = jnp.minimum(a, b)
    m1 = None
    cum = None
    kth = None
    for i in range(TOP_K):
        front = parts[0]
        m = jnp.max(front, axis=1, keepdims=True)
        eq = front == m
        c = jnp.sum(eq.astype(s.dtype), axis=1, keepdims=True)
        if i == 0:
            m1 = m
            kth = m
            cum = c
        else:
            take = jnp.logical_and(cum < TOP_K, cum + c >= TOP_K)
            kth = jnp.where(take, m, kth)
            cum = cum + c
        for j in range(_NPARTS - 1 - i):
            parts[j] = jnp.where(eq, parts[j + 1], parts[j])
        if i < TOP_K - 1:
            parts[_NPARTS - 1 - i] = jnp.where(eq, -jnp.inf,
                                               parts[_NPARTS - 1 - i])
    return kth, m1


def _sparse_softmax_rows(s):
    kth, m1 = _topk_rows(s)
    e = jnp.where(s >= kth, jnp.exp((s - m1) * SCALE), 0.0)
    return e / jnp.sum(e, axis=1, keepdims=True)


def _fused_kernel(x_ref, M_ref, Wq_ref, Wo_ref, We_ref, Wa_ref, gate_ref,
                  lnw_ref, lnb_ref, be_ref, ba_ref,
                  out_ref, rep_ref, Mnew_ref,
                  aw_ref, er_ref, ad_ref):
    x = x_ref[0]
    Mb = M_ref[0]
    Mb16 = _bf16(Mb)
    q = jax.lax.dot_general(x, Wq_ref[...], CDIMS,
                            preferred_element_type=jnp.float32)
    s = jax.lax.dot_general(q, Mb, CDIMS,
                            preferred_element_type=jnp.float32)
    addr = _sparse_softmax_rows(s)
    r = jax.lax.dot_general(_bf16(addr), Mb16, (((1,), (0,)), ((), ())),
                            preferred_element_type=jnp.float32)
    mu = jnp.mean(r, axis=1, keepdims=True)
    var = jnp.mean((r - mu) ** 2, axis=1, keepdims=True)
    ln = (r - mu) * jax.lax.rsqrt(var + EPS) * lnw_ref[...] + lnb_ref[...]
    out_ref[0] = jax.lax.dot_general(_bf16(ln), _bf16(Wo_ref[...]), CDIMS,
                                     preferred_element_type=jnp.float32)
    rep_ref[0] = jax.nn.sigmoid(gate_ref[...]) * r

    @pl.when(pl.program_id(1) == 0)
    def _write_addr():
        q_last = q[TL - 1:TL, :]
        s_col = jax.lax.dot_general(Mb, q_last, CDIMS,
                                    preferred_element_type=jnp.float32)
        kth, m1 = _topk_rows(s_col.T)
        e = jnp.where(s_col >= kth, jnp.exp((s_col - m1) * SCALE), 0.0)
        aw_ref[...] = e / jnp.sum(e, axis=0, keepdims=True)
        xl = x[TL - 1:TL, :]
        er_ref[...] = jax.nn.sigmoid(
            jax.lax.dot_general(xl, We_ref[...], CDIMS,
                                preferred_element_type=jnp.float32)
            + be_ref[...])
        ad_ref[...] = jnp.tanh(
            jax.lax.dot_general(xl, Wa_ref[...], CDIMS,
                                preferred_element_type=jnp.float32)
            + ba_ref[...])

    li = pl.program_id(1)
    aw = aw_ref[pl.ds(li * MS, MS), :]
    Ms = M_ref[0, pl.ds(li * MS, MS), :]
    Mnew_ref[0] = Ms * (1.0 - aw * er_ref[...]) + aw * ad_ref[...]


def kernel(x, M, W_q, W_e, b_e, W_a, b_a, W_o, r_gate, ln_w, ln_b):
    gate2 = r_gate.reshape(1, D)
    lnw2 = ln_w.reshape(1, D)
    lnb2 = ln_b.reshape(1, D)
    be2 = b_e.reshape(1, D)
    ba2 = b_a.reshape(1, D)

    wcell = lambda b, l: (0, 0)
    rev = lambda b, l: (b, NL - 1 - l, 0)
    out, rep, M_new = pl.pallas_call(
        _fused_kernel,
        grid=(B, NL),
        in_specs=[
            pl.BlockSpec((1, TL, D), rev),
            pl.BlockSpec((1, SLOTS, D), lambda b, l: (b, 0, 0)),
            pl.BlockSpec((D, D), wcell),
            pl.BlockSpec((D, D), wcell),
            pl.BlockSpec((D, D), wcell),
            pl.BlockSpec((D, D), wcell),
            pl.BlockSpec((1, D), wcell),
            pl.BlockSpec((1, D), wcell),
            pl.BlockSpec((1, D), wcell),
            pl.BlockSpec((1, D), wcell),
            pl.BlockSpec((1, D), wcell),
        ],
        out_specs=[
            pl.BlockSpec((1, TL, D), rev),
            pl.BlockSpec((1, TL, D), rev),
            pl.BlockSpec((1, MS, D), lambda b, l: (b, l, 0)),
        ],
        out_shape=[
            jax.ShapeDtypeStruct((B, L, D), jnp.float32),
            jax.ShapeDtypeStruct((B, L, D), jnp.float32),
            jax.ShapeDtypeStruct((B, SLOTS, D), jnp.float32),
        ],
        scratch_shapes=[
            pltpu.VMEM((SLOTS, 1), jnp.float32),
            pltpu.VMEM((1, D), jnp.float32),
            pltpu.VMEM((1, D), jnp.float32),
        ],
    )(x, M, W_q, W_o, W_e, W_a, gate2, lnw2, lnb2, be2, ba2)

    return out, rep, M_new
